# baseline (speedup 1.0000x reference)
import functools

import jax
import jax.numpy as jnp
from jax import lax
from jax.experimental import pallas as pl
from jax.experimental.pallas import tpu as pltpu
from jax.experimental.pallas import tpu_sc as plsc

_B = 16384
_D = 16

_NC = 2
_NS = 16
_NW = _NC * _NS
_BPW = _B // _NW
_CHUNK = 128
_NCHUNK = _BPW // _CHUNK


def _sc_gather6(u, p, n, t_gu, t_gi, t_mu, t_mi):
    mesh = plsc.VectorSubcoreMesh(core_axis_name="c", subcore_axis_name="s")
    out_t = tuple(jax.ShapeDtypeStruct((_B, _D), jnp.float32) for _ in range(6))
    scratch = (
        [pltpu.VMEM((_BPW,), jnp.int32) for _ in range(3)]
        + [pltpu.VMEM((_BPW, _D), jnp.float32) for _ in range(6)]
        + [pltpu.SemaphoreType.DMA]
    )

    @functools.partial(
        pl.kernel, mesh=mesh, out_type=out_t, scratch_types=scratch,
        compiler_params=pltpu.CompilerParams(use_tc_tiling_on_sc=False))
    def body(u_h, p_h, n_h, tgu_h, tgi_h, tmu_h, tmi_h,
             o_gu, o_gp, o_gn, o_mu, o_mp, o_mn,
             uv, pv, nv, gu_v, gp_v, gn_v, mu_v, mp_v, mn_v, sem):
        wid = lax.axis_index("s") * _NC + lax.axis_index("c")
        base = wid * _BPW
        pltpu.sync_copy(u_h.at[pl.ds(base, _BPW)], uv)
        pltpu.sync_copy(p_h.at[pl.ds(base, _BPW)], pv)
        pltpu.sync_copy(n_h.at[pl.ds(base, _BPW)], nv)
        copies = []
        for j in range(_NCHUNK):
            sl = pl.ds(j * _CHUNK, _CHUNK)
            for tab, iv, dst in ((tgu_h, uv, gu_v), (tgi_h, pv, gp_v),
                                 (tgi_h, nv, gn_v), (tmu_h, uv, mu_v),
                                 (tmi_h, pv, mp_v), (tmi_h, nv, mn_v)):
                copies.append(pltpu.async_copy(tab.at[iv.at[sl]],
                                               dst.at[sl], sem))
        for c in copies:
            c.wait()
        for src, out in ((gu_v, o_gu), (gp_v, o_gp), (gn_v, o_gn),
                         (mu_v, o_mu), (mp_v, o_mp), (mn_v, o_mn)):
            pltpu.sync_copy(src, out.at[pl.ds(base, _BPW)])

    return body(u, p, n, t_gu, t_gi, t_mu, t_mi)


_BLK = 2048


def _r16(x):
    return x.astype(jnp.bfloat16).astype(jnp.float32)


def _tc_body(gu_r, gp_r, gn_r, mu_r, mp_r, mn_r,
             w1_r, b1_r, w2_r, b2_r, w3_r, b3_r, wdg_r, wdm_r, bd_r, out_r):
    f32 = jnp.float32
    hi = lax.Precision.HIGHEST
    gu = gu_r[...]
    gmf_p = jax.nn.sigmoid(gu * gp_r[...])
    gmf_n = jax.nn.sigmoid(gu * gn_r[...])

    w1 = _r16(w1_r[...])
    w1a, w1b = w1[:_D], w1[_D:]
    b1 = b1_r[...]
    w2 = _r16(w2_r[...])
    b2 = b2_r[...]
    w3 = _r16(w3_r[...])
    b3 = b3_r[...]
    mu = _r16(mu_r[...])
    u_part = jnp.dot(mu, w1a, preferred_element_type=f32, precision=hi)

    def dnn(u_part, xi):
        h = u_part + jnp.dot(_r16(xi), w1b, preferred_element_type=f32,
                             precision=hi) + b1
        h = jnp.maximum(h, 0.0)
        h = jnp.maximum(jnp.dot(_r16(h), w2, preferred_element_type=f32,
                                precision=hi) + b2, 0.0)
        h = jnp.maximum(jnp.dot(_r16(h), w3, preferred_element_type=f32,
                                precision=hi) + b3, 0.0)
        return h

    hp = dnn(u_part, mp_r[...])
    hn = dnn(u_part, mn_r[...])

    wdg = _r16(wdg_r[...])
    wdm = _r16(wdm_r[...])
    bd = bd_r[...]
    pos = (jnp.sum(_r16(gmf_p) * wdg, axis=1, keepdims=True)
           + jnp.sum(_r16(hp) * wdm, axis=1, keepdims=True) + bd)
    neg = (jnp.sum(_r16(gmf_n) * wdg, axis=1, keepdims=True)
           + jnp.sum(_r16(hn) * wdm, axis=1, keepdims=True) + bd)
    out_r[...] = jnp.concatenate([pos, neg], axis=1)


def _tc_mlp(gu, gp, gn, mu, mp_, mn, w1, b1, w2, b2, w3, b3, wd, bd):
    grid = (_B // _BLK,)
    row_spec = pl.BlockSpec((_BLK, _D), lambda i: (i, 0))
    full = lambda s: pl.BlockSpec(s, lambda i: (0, 0))
    return pl.pallas_call(
        _tc_body,
        grid=grid,
        in_specs=[row_spec] * 6 + [
            full((2 * _D, 64)), full((1, 64)),
            full((64, 16)), full((1, 16)),
            full((16, 8)), full((1, 8)),
            full((1, _D)), full((1, 8)), full((1, 1)),
        ],
        out_specs=pl.BlockSpec((_BLK, 2), lambda i: (i, 0)),
        out_shape=jax.ShapeDtypeStruct((_B, 2), jnp.float32),
    )(gu, gp, gn, mu, mp_, mn,
      w1, b1.reshape(1, 64), w2, b2.reshape(1, 16), w3, b3.reshape(1, 8),
      wd[:_D].reshape(1, _D), wd[_D:].reshape(1, 8), bd.reshape(1, 1))


def kernel(user_inputs, pos_inputs, neg_inputs,
           gmf_user_table, gmf_item_table, mlp_user_table, mlp_item_table,
           w1, b1, w2, b2, w3, b3, wd, bd):
    u = user_inputs.reshape(_B).astype(jnp.int32)
    p = pos_inputs.reshape(_B).astype(jnp.int32)
    n = neg_inputs.reshape(_B).astype(jnp.int32)
    gu, gp, gn, mu, mp_, mn = _sc_gather6(
        u, p, n, gmf_user_table, gmf_item_table, mlp_user_table,
        mlp_item_table)
    return _tc_mlp(gu, gp, gn, mu, mp_, mn,
                   w1, b1, w2, b2, w3, b3, wd, bd)
